# baseline (speedup 1.0000x reference)
import functools

import numpy as np
import jax
import jax.numpy as jnp
from jax import lax
from jax.experimental import pallas as pl
from jax.experimental.pallas import tpu as pltpu
from jax.experimental.pallas import tpu_sc as plsc

N = 10000
D = 128
E = 320000

NC = 2
NS = 16
NW = NC * NS

CH = 128
N_PAD = 10112
RT = N_PAD // NS
EPW = 10112
E_PAD = EPW * NW
NCHUNK = EPW // CH

DW = 16


@functools.lru_cache(maxsize=None)
def _make_agg(with_deg):
    mesh = plsc.VectorSubcoreMesh(
        core_axis_name="c", subcore_axis_name="s", num_cores=NC, num_subcores=NS)

    out_type = [jax.ShapeDtypeStruct((NC, N_PAD, D), jnp.float32)]
    scratch = [
        pltpu.VMEM((CH,), jnp.int32),
        pltpu.VMEM((CH,), jnp.int32),
        pltpu.VMEM((CH, D), jnp.float32),
        pltpu.VMEM_SHARED((N_PAD, D), jnp.float32),
        pltpu.SemaphoreType.DMA,
    ]
    if with_deg:
        out_type.append(jax.ShapeDtypeStruct((NC, N_PAD, DW), jnp.float32))
        scratch += [
            pltpu.VMEM((CH, DW), jnp.float32),
            pltpu.VMEM_SHARED((N_PAD, DW), jnp.float32),
        ]

    def agg(*refs):
        if with_deg:
            (tab_hbm, src_hbm, dst_hbm, zeros_hbm, z16_hbm, ones_hbm,
             out_hbm, deg_out_hbm, src_v, dst_v, rows_v, acc, sem,
             ones_v, acc16) = refs
        else:
            (tab_hbm, src_hbm, dst_hbm, zeros_hbm,
             out_hbm, src_v, dst_v, rows_v, acc, sem) = refs
        c = lax.axis_index("c")
        s = lax.axis_index("s")
        wid = c * jnp.int32(NS) + s
        row0 = s * jnp.int32(RT)
        pltpu.sync_copy(zeros_hbm, acc.at[pl.ds(row0, RT)])
        if with_deg:
            pltpu.sync_copy(z16_hbm, acc16.at[pl.ds(row0, RT)])
            pltpu.sync_copy(ones_hbm, ones_v)
        plsc.subcore_barrier()
        base = wid * jnp.int32(EPW)

        def body(j, carry):
            off = base + j * jnp.int32(CH)
            pltpu.sync_copy(src_hbm.at[pl.ds(off, CH)], src_v)
            pltpu.sync_copy(dst_hbm.at[pl.ds(off, CH)], dst_v)
            pltpu.async_copy(tab_hbm.at[src_v], rows_v, sem).wait()
            pltpu.sync_copy(rows_v, acc.at[dst_v], add=True)
            if with_deg:
                pltpu.sync_copy(ones_v, acc16.at[dst_v], add=True)
            return carry

        lax.fori_loop(jnp.int32(0), jnp.int32(NCHUNK), body, jnp.int32(0))
        plsc.subcore_barrier()
        pltpu.sync_copy(acc.at[pl.ds(row0, RT)], out_hbm.at[c, pl.ds(row0, RT)])
        if with_deg:
            pltpu.sync_copy(acc16.at[pl.ds(row0, RT)],
                            deg_out_hbm.at[c, pl.ds(row0, RT)])

    return pl.kernel(agg, out_type=out_type, mesh=mesh, scratch_types=scratch)


def _make_layer(relu, emit_inv):
    B = 1000
    grid = (N // B,)

    def body(*refs):
        if emit_inv:
            x_ref, acc_ref, d0_ref, d1_ref, ws_ref, wn_ref, b_ref, o_ref, inv_ref = refs
            deg = d0_ref[:, 0:1] + d1_ref[:, 0:1]
            inv = 1.0 / jnp.maximum(deg, 1.0)
            inv_ref[...] = jnp.broadcast_to(inv, (B, 8))
        else:
            x_ref, acc_ref, inv8_ref, ws_ref, wn_ref, b_ref, o_ref = refs
            inv = inv8_ref[:, 0:1]
        feats = acc_ref[0] + acc_ref[1]
        nb = feats * inv
        h = (jnp.dot(x_ref[...], ws_ref[...], preferred_element_type=jnp.float32)
             + jnp.dot(nb, wn_ref[...], preferred_element_type=jnp.float32)
             + b_ref[...])
        if relu:
            h = jnp.maximum(h, 0.0)
        o_ref[...] = h

    z = np.int32(0)
    x_spec = pl.BlockSpec((B, D), lambda i: (i, z))
    acc_spec = pl.BlockSpec((NC, B, D), lambda i: (z, i, z))
    w_spec = pl.BlockSpec((D, D), lambda i: (z, z))
    b_spec = pl.BlockSpec((1, D), lambda i: (z, z))
    inv_spec = pl.BlockSpec((B, 8), lambda i: (i, z))

    in_specs = ([x_spec, acc_spec]
                + ([inv_spec, inv_spec] if emit_inv else [inv_spec])
                + [w_spec, w_spec, b_spec])
    out_shape = [jax.ShapeDtypeStruct((N, D), jnp.float32)]
    out_specs = [x_spec]
    if emit_inv:
        out_shape.append(jax.ShapeDtypeStruct((N, 8), jnp.float32))
        out_specs.append(inv_spec)

    return pl.pallas_call(
        body, grid=grid, in_specs=in_specs,
        out_specs=out_specs if len(out_specs) > 1 else out_specs[0],
        out_shape=out_shape if len(out_shape) > 1 else out_shape[0])


_layer0 = _make_layer(relu=True, emit_inv=True)
_layer1 = _make_layer(relu=True, emit_inv=False)
_layer2 = _make_layer(relu=False, emit_inv=False)


def _fold_bn(Ws, Wn, b, gamma, beta, eps=1e-5):
    s = gamma * np.float32(1.0 / np.sqrt(1.0 + eps))
    return Ws * s[None, :], Wn * s[None, :], (b * s + beta).reshape(1, D)


def kernel(x, edge_index, W_self0, W_neigh0, b0, gamma0, beta0,
           W_self1, W_neigh1, b1, gamma1, beta1, W_self2, W_neigh2, b2):
    x = x.astype(jnp.float32)
    src = edge_index[0].astype(jnp.int32)
    dst = edge_index[1].astype(jnp.int32)
    pad = np.full((E_PAD - E,), N, dtype=np.int32)
    src_p = jnp.concatenate([src, pad])
    dst_p = jnp.concatenate([dst, pad])

    row_pad = np.zeros((N_PAD - N, D), np.float32)
    z = np.zeros((RT, D), np.float32)
    z16 = np.zeros((RT, DW), np.float32)
    ones = np.ones((CH, DW), np.float32)

    Ws0, Wn0, B0 = _fold_bn(W_self0, W_neigh0, b0, gamma0, beta0)
    Ws1, Wn1, B1 = _fold_bn(W_self1, W_neigh1, b1, gamma1, beta1)
    B2 = b2.reshape(1, D)

    x_pad = jnp.concatenate([x, row_pad], axis=0)
    (acc0,) = _make_agg(False)(x_pad, src_p, dst_p, z)
    ones_tab = np.ones((N_PAD, D), np.float32)
    src_ones = np.full((E_PAD,), N, dtype=np.int32)
    (degp,) = _make_agg(False)(ones_tab, src_ones, dst_p, z)
    d0 = degp[0, :N, 0:8]
    d1 = degp[1, :N, 0:8]
    h1, inv8 = _layer0(x, acc0, d0, d1, Ws0, Wn0, B0)

    (acc1,) = _make_agg(False)(jnp.concatenate([h1, row_pad], axis=0), src_p, dst_p, z)
    h2 = _layer1(h1, acc1, inv8, Ws1, Wn1, B1)

    (acc2,) = _make_agg(False)(jnp.concatenate([h2, row_pad], axis=0), src_p, dst_p, z)
    h3 = _layer2(h2, acc2, inv8, W_self2, W_neigh2, B2)
    return h3


# speedup vs baseline: 7.5883x; 7.5883x over previous
import functools

import numpy as np
import jax
import jax.numpy as jnp
from jax import lax
from jax.experimental import pallas as pl
from jax.experimental.pallas import tpu as pltpu
from jax.experimental.pallas import tpu_sc as plsc

N = 10000
D = 128
E = 320000

NC = 2
NS = 16
NW = NC * NS

CH = 128
N_PAD = 10112
RT = N_PAD // NS
EPW = 10112
E_PAD = EPW * NW
NCHUNK = EPW // CH

DW = 16


@functools.lru_cache(maxsize=None)
def _make_agg(with_deg):
    mesh = plsc.VectorSubcoreMesh(
        core_axis_name="c", subcore_axis_name="s", num_cores=NC, num_subcores=NS)

    out_type = [jax.ShapeDtypeStruct((NC, N_PAD, D), jnp.float32)]
    scratch = [
        pltpu.VMEM((CH,), jnp.int32),
        pltpu.VMEM((CH,), jnp.int32),
        pltpu.VMEM((CH, D), jnp.float32),
        pltpu.VMEM_SHARED((N_PAD, D), jnp.float32),
        pltpu.SemaphoreType.DMA,
    ]
    if with_deg:
        out_type.append(jax.ShapeDtypeStruct((NC, N_PAD, DW), jnp.float32))
        scratch += [
            pltpu.VMEM((CH, DW), jnp.float32),
            pltpu.VMEM_SHARED((N_PAD, DW), jnp.float32),
        ]

    def agg(*refs):
        if with_deg:
            (tab_hbm, src_hbm, dst_hbm, zeros_hbm, z16_hbm, ones_hbm,
             out_hbm, deg_out_hbm, src_v, dst_v, rows_v, acc, sem,
             ones_v, acc16) = refs
        else:
            (tab_hbm, src_hbm, dst_hbm, zeros_hbm,
             out_hbm, src_v, dst_v, rows_v, acc, sem) = refs
        c = lax.axis_index("c")
        s = lax.axis_index("s")
        wid = c * jnp.int32(NS) + s
        row0 = s * jnp.int32(RT)
        pltpu.sync_copy(zeros_hbm, acc.at[pl.ds(row0, RT)])
        if with_deg:
            pltpu.sync_copy(z16_hbm, acc16.at[pl.ds(row0, RT)])
            pltpu.sync_copy(ones_hbm, ones_v)
        plsc.subcore_barrier()
        base = wid * jnp.int32(EPW)

        def body(j, carry):
            off = base + j * jnp.int32(CH)
            pltpu.sync_copy(src_hbm.at[pl.ds(off, CH)], src_v)
            pltpu.sync_copy(dst_hbm.at[pl.ds(off, CH)], dst_v)
            pltpu.async_copy(tab_hbm.at[src_v], rows_v, sem).wait()
            pltpu.sync_copy(rows_v, acc.at[dst_v], add=True)
            if with_deg:
                pltpu.sync_copy(ones_v, acc16.at[dst_v], add=True)
            return carry

        lax.fori_loop(jnp.int32(0), jnp.int32(NCHUNK), body, jnp.int32(0))
        plsc.subcore_barrier()
        pltpu.sync_copy(acc.at[pl.ds(row0, RT)], out_hbm.at[c, pl.ds(row0, RT)])
        if with_deg:
            pltpu.sync_copy(acc16.at[pl.ds(row0, RT)],
                            deg_out_hbm.at[c, pl.ds(row0, RT)])

    return pl.kernel(agg, out_type=out_type, mesh=mesh, scratch_types=scratch)


def _make_layer(relu, emit_inv):
    B = 1000
    grid = (N // B,)

    def body(*refs):
        if emit_inv:
            x_ref, acc_ref, d0_ref, d1_ref, ws_ref, wn_ref, b_ref, o_ref, inv_ref = refs
            deg = d0_ref[:, 0:1] + d1_ref[:, 0:1]
            inv = 1.0 / jnp.maximum(deg, 1.0)
            inv_ref[...] = jnp.broadcast_to(inv, (B, 8))
        else:
            x_ref, acc_ref, inv8_ref, ws_ref, wn_ref, b_ref, o_ref = refs
            inv = inv8_ref[:, 0:1]
        feats = acc_ref[0] + acc_ref[1]
        nb = feats * inv
        h = (jnp.dot(x_ref[...], ws_ref[...], preferred_element_type=jnp.float32)
             + jnp.dot(nb, wn_ref[...], preferred_element_type=jnp.float32)
             + b_ref[...])
        if relu:
            h = jnp.maximum(h, 0.0)
        o_ref[...] = h

    z = np.int32(0)
    x_spec = pl.BlockSpec((B, D), lambda i: (i, z))
    acc_spec = pl.BlockSpec((NC, B, D), lambda i: (z, i, z))
    w_spec = pl.BlockSpec((D, D), lambda i: (z, z))
    b_spec = pl.BlockSpec((1, D), lambda i: (z, z))
    inv_spec = pl.BlockSpec((B, 8), lambda i: (i, z))

    in_specs = ([x_spec, acc_spec]
                + ([inv_spec, inv_spec] if emit_inv else [inv_spec])
                + [w_spec, w_spec, b_spec])
    out_shape = [jax.ShapeDtypeStruct((N, D), jnp.float32)]
    out_specs = [x_spec]
    if emit_inv:
        out_shape.append(jax.ShapeDtypeStruct((N, 8), jnp.float32))
        out_specs.append(inv_spec)

    return pl.pallas_call(
        body, grid=grid, in_specs=in_specs,
        out_specs=out_specs if len(out_specs) > 1 else out_specs[0],
        out_shape=out_shape if len(out_shape) > 1 else out_shape[0])


_layer0 = _make_layer(relu=True, emit_inv=True)
_layer1 = _make_layer(relu=True, emit_inv=False)
_layer2 = _make_layer(relu=False, emit_inv=False)


def _fold_bn(Ws, Wn, b, gamma, beta, eps=1e-5):
    s = gamma * np.float32(1.0 / np.sqrt(1.0 + eps))
    return Ws * s[None, :], Wn * s[None, :], (b * s + beta).reshape(1, D)


def kernel(x, edge_index, W_self0, W_neigh0, b0, gamma0, beta0,
           W_self1, W_neigh1, b1, gamma1, beta1, W_self2, W_neigh2, b2):
    x = x.astype(jnp.float32)
    src = edge_index[0].astype(jnp.int32)
    dst = edge_index[1].astype(jnp.int32)
    pad = np.full((E_PAD - E,), N, dtype=np.int32)
    src_p = jnp.concatenate([src, pad])
    dst_p = jnp.concatenate([dst, pad])

    row_pad = np.zeros((N_PAD - N, D), np.float32)
    z = np.zeros((RT, D), np.float32)
    z16 = np.zeros((RT, DW), np.float32)
    ones = np.ones((CH, DW), np.float32)

    Ws0, Wn0, B0 = _fold_bn(W_self0, W_neigh0, b0, gamma0, beta0)
    Ws1, Wn1, B1 = _fold_bn(W_self1, W_neigh1, b1, gamma1, beta1)
    B2 = b2.reshape(1, D)

    x_pad = jnp.concatenate([x, row_pad], axis=0)
    (acc0,) = _make_agg(False)(x_pad, src_p, dst_p, z)
    ones_tab = np.ones((N_PAD, D), np.float32)
    src_ones = (np.arange(E_PAD, dtype=np.int32) % np.int32(N_PAD - N)) + np.int32(N)
    (degp,) = _make_agg(False)(ones_tab, src_ones, dst_p, z)
    d0 = degp[0, :N, 0:8]
    d1 = degp[1, :N, 0:8]
    h1, inv8 = _layer0(x, acc0, d0, d1, Ws0, Wn0, B0)

    (acc1,) = _make_agg(False)(jnp.concatenate([h1, row_pad], axis=0), src_p, dst_p, z)
    h2 = _layer1(h1, acc1, inv8, Ws1, Wn1, B1)

    (acc2,) = _make_agg(False)(jnp.concatenate([h2, row_pad], axis=0), src_p, dst_p, z)
    h3 = _layer2(h2, acc2, inv8, W_self2, W_neigh2, B2)
    return h3


# speedup vs baseline: 26.8025x; 3.5321x over previous
import functools

import numpy as np
import jax
import jax.numpy as jnp
from jax import lax
from jax.experimental import pallas as pl
from jax.experimental.pallas import tpu as pltpu
from jax.experimental.pallas import tpu_sc as plsc

N = 10000
D = 128
E = 320000

NC = 2
NS = 16
NW = NC * NS

CH = 128
NBUF = 2
N_PAD = 10112
RT = N_PAD // NS
NCHUNK = 80
NSTAGE = 2
SCHUNK = NCHUNK // NSTAGE
EPW = NCHUNK * CH
E_PAD = EPW * NW
NDUMMY = N_PAD - N

DW = 16


@functools.lru_cache(maxsize=None)
def _make_agg():
    mesh = plsc.VectorSubcoreMesh(
        core_axis_name="c", subcore_axis_name="s", num_cores=NC, num_subcores=NS)

    out_type = [jax.ShapeDtypeStruct((NC, N_PAD, D), jnp.float32)]
    scratch = (
        [pltpu.VMEM((SCHUNK, CH), jnp.int32),
         pltpu.VMEM((SCHUNK, CH), jnp.int32)]
        + [pltpu.VMEM((CH, D), jnp.float32) for _ in range(NBUF)]
        + [pltpu.VMEM_SHARED((N_PAD, D), jnp.float32)]
        + [pltpu.SemaphoreType.DMA for _ in range(NBUF)]
    )

    def agg(tab_hbm, src_hbm, dst_hbm, zeros_hbm, out_hbm, src_v, dst_v, *rest):
        rows = rest[:NBUF]
        acc = rest[NBUF]
        sems = rest[NBUF + 1:]
        c = lax.axis_index("c")
        s = lax.axis_index("s")
        wid = c * jnp.int32(NS) + s
        row0 = s * jnp.int32(RT)
        pltpu.sync_copy(zeros_hbm, acc.at[pl.ds(row0, RT)])
        plsc.subcore_barrier()

        for t in range(NSTAGE):
            c0 = jnp.int32(t * SCHUNK)
            pltpu.sync_copy(src_hbm.at[wid, pl.ds(c0, SCHUNK)], src_v)
            pltpu.sync_copy(dst_hbm.at[wid, pl.ds(c0, SCHUNK)], dst_v)
            for b in range(NBUF):
                pltpu.async_copy(
                    tab_hbm.at[src_v.at[jnp.int32(b)]], rows[b], sems[b])

            def body(i, carry):
                j0 = i * jnp.int32(NBUF)
                for b in range(NBUF):
                    j = j0 + jnp.int32(b)
                    pltpu.make_async_copy(
                        tab_hbm.at[src_v.at[j]], rows[b], sems[b]).wait()
                    pltpu.sync_copy(rows[b], acc.at[dst_v.at[j]], add=True)
                    pltpu.async_copy(
                        tab_hbm.at[src_v.at[j + jnp.int32(NBUF)]],
                        rows[b], sems[b])
                return carry

            lax.fori_loop(jnp.int32(0), jnp.int32(SCHUNK // NBUF - 1), body,
                          jnp.int32(0))
            jlast = jnp.int32(SCHUNK - NBUF)
            for b in range(NBUF):
                j = jlast + jnp.int32(b)
                pltpu.make_async_copy(
                    tab_hbm.at[src_v.at[j]], rows[b], sems[b]).wait()
                pltpu.sync_copy(rows[b], acc.at[dst_v.at[j]], add=True)

        plsc.subcore_barrier()
        pltpu.sync_copy(acc.at[pl.ds(row0, RT)], out_hbm.at[c, pl.ds(row0, RT)])

    return pl.kernel(agg, out_type=out_type, mesh=mesh, scratch_types=scratch)


@functools.lru_cache(maxsize=None)
def _make_deg():
    mesh = plsc.VectorSubcoreMesh(
        core_axis_name="c", subcore_axis_name="s", num_cores=NC, num_subcores=NS)

    out_type = [jax.ShapeDtypeStruct((NC, N_PAD, D), jnp.float32)]
    scratch = [
        pltpu.VMEM((NCHUNK, CH), jnp.int32),
        pltpu.VMEM((CH, D), jnp.float32),
        pltpu.VMEM_SHARED((N_PAD, D), jnp.float32),
    ]

    def deg(dst_hbm, zeros_hbm, ones_hbm, out_hbm, dst_v, ones_v, acc):
        c = lax.axis_index("c")
        s = lax.axis_index("s")
        wid = c * jnp.int32(NS) + s
        row0 = s * jnp.int32(RT)
        pltpu.sync_copy(zeros_hbm, acc.at[pl.ds(row0, RT)])
        pltpu.sync_copy(dst_hbm.at[wid], dst_v)
        pltpu.sync_copy(ones_hbm, ones_v)
        plsc.subcore_barrier()

        def body(j, carry):
            pltpu.sync_copy(ones_v, acc.at[dst_v.at[j]], add=True)
            return carry

        lax.fori_loop(jnp.int32(0), jnp.int32(NCHUNK), body, jnp.int32(0))
        plsc.subcore_barrier()
        pltpu.sync_copy(acc.at[pl.ds(row0, RT)],
                        out_hbm.at[c, pl.ds(row0, RT)])

    return pl.kernel(deg, out_type=out_type, mesh=mesh, scratch_types=scratch)


def _make_layer(relu, emit_inv):
    B = 1000
    grid = (N // B,)

    def body(*refs):
        if emit_inv:
            x_ref, acc_ref, d0_ref, d1_ref, ws_ref, wn_ref, b_ref, o_ref, inv_ref = refs
            deg = d0_ref[:, 0:1] + d1_ref[:, 0:1]
            inv = 1.0 / jnp.maximum(deg, 1.0)
            inv_ref[...] = jnp.broadcast_to(inv, (B, 8))
        else:
            x_ref, acc_ref, inv8_ref, ws_ref, wn_ref, b_ref, o_ref = refs
            inv = inv8_ref[:, 0:1]
        feats = acc_ref[0] + acc_ref[1]
        nb = feats * inv
        h = (jnp.dot(x_ref[...], ws_ref[...], preferred_element_type=jnp.float32)
             + jnp.dot(nb, wn_ref[...], preferred_element_type=jnp.float32)
             + b_ref[...])
        if relu:
            h = jnp.maximum(h, 0.0)
        o_ref[...] = h

    z = np.int32(0)
    x_spec = pl.BlockSpec((B, D), lambda i: (i, z))
    acc_spec = pl.BlockSpec((NC, B, D), lambda i: (z, i, z))
    w_spec = pl.BlockSpec((D, D), lambda i: (z, z))
    b_spec = pl.BlockSpec((1, D), lambda i: (z, z))
    inv_spec = pl.BlockSpec((B, 8), lambda i: (i, z))

    in_specs = ([x_spec, acc_spec]
                + ([inv_spec, inv_spec] if emit_inv else [inv_spec])
                + [w_spec, w_spec, b_spec])
    out_shape = [jax.ShapeDtypeStruct((N, D), jnp.float32)]
    out_specs = [x_spec]
    if emit_inv:
        out_shape.append(jax.ShapeDtypeStruct((N, 8), jnp.float32))
        out_specs.append(inv_spec)

    return pl.pallas_call(
        body, grid=grid, in_specs=in_specs,
        out_specs=out_specs if len(out_specs) > 1 else out_specs[0],
        out_shape=out_shape if len(out_shape) > 1 else out_shape[0])


_layer0 = _make_layer(relu=True, emit_inv=True)
_layer1 = _make_layer(relu=True, emit_inv=False)
_layer2 = _make_layer(relu=False, emit_inv=False)


def _fold_bn(Ws, Wn, b, gamma, beta, eps=1e-5):
    s = gamma * np.float32(1.0 / np.sqrt(1.0 + eps))
    return Ws * s[None, :], Wn * s[None, :], (b * s + beta).reshape(1, D)


def kernel(x, edge_index, W_self0, W_neigh0, b0, gamma0, beta0,
           W_self1, W_neigh1, b1, gamma1, beta1, W_self2, W_neigh2, b2):
    x = x.astype(jnp.float32)
    src = edge_index[0].astype(jnp.int32)
    dst = edge_index[1].astype(jnp.int32)
    pad = (np.arange(E_PAD - E, dtype=np.int32) % np.int32(NDUMMY)) + np.int32(N)
    src_p = jnp.concatenate([src, pad]).reshape(NW, NCHUNK, CH)
    dst_p = jnp.concatenate([dst, pad]).reshape(NW, NCHUNK, CH)

    row_pad = np.zeros((N_PAD - N, D), np.float32)
    z = np.zeros((RT, D), np.float32)

    Ws0, Wn0, B0 = _fold_bn(W_self0, W_neigh0, b0, gamma0, beta0)
    Ws1, Wn1, B1 = _fold_bn(W_self1, W_neigh1, b1, gamma1, beta1)
    B2 = b2.reshape(1, D)

    x_pad = jnp.concatenate([x, row_pad], axis=0)
    (acc0,) = _make_agg()(x_pad, src_p, dst_p, z)
    ones = np.ones((CH, D), np.float32)
    (degp,) = _make_deg()(dst_p, z, ones)
    d0 = degp[0, :N, 0:8]
    d1 = degp[1, :N, 0:8]
    h1, inv8 = _layer0(x, acc0, d0, d1, Ws0, Wn0, B0)

    (acc1,) = _make_agg()(jnp.concatenate([h1, row_pad], axis=0), src_p, dst_p, z)
    h2 = _layer1(h1, acc1, inv8, Ws1, Wn1, B1)

    (acc2,) = _make_agg()(jnp.concatenate([h2, row_pad], axis=0), src_p, dst_p, z)
    h3 = _layer2(h2, acc2, inv8, W_self2, W_neigh2, B2)
    return h3


# speedup vs baseline: 26.9005x; 1.0037x over previous
import functools

import numpy as np
import jax
import jax.numpy as jnp
from jax import lax
from jax.experimental import pallas as pl
from jax.experimental.pallas import tpu as pltpu
from jax.experimental.pallas import tpu_sc as plsc

N = 10000
D = 128
E = 320000

NC = 2
NS = 16
NW = NC * NS

CH = 128
NBUF = 2
N_PAD = 10112
RT = N_PAD // NS
NCHUNK = 80
NSTAGE = 2
SCHUNK = NCHUNK // NSTAGE
EPW = NCHUNK * CH
E_PAD = EPW * NW
NDUMMY = N_PAD - N


@functools.lru_cache(maxsize=None)
def _make_agg():
    mesh = plsc.VectorSubcoreMesh(
        core_axis_name="c", subcore_axis_name="s", num_cores=NC, num_subcores=NS)

    out_type = [jax.ShapeDtypeStruct((NC, N_PAD, D), jnp.float32)]
    scratch = (
        [pltpu.VMEM((SCHUNK, CH), jnp.int32),
         pltpu.VMEM((SCHUNK, CH), jnp.int32)]
        + [pltpu.VMEM((CH, D), jnp.float32) for _ in range(NBUF)]
        + [pltpu.VMEM_SHARED((N_PAD, D), jnp.float32)]
        + [pltpu.SemaphoreType.DMA for _ in range(NBUF)]
    )

    def agg(tab_hbm, src_hbm, dst_hbm, zeros_hbm, out_hbm, src_v, dst_v, *rest):
        rows = rest[:NBUF]
        acc = rest[NBUF]
        sems = rest[NBUF + 1:]
        c = lax.axis_index("c")
        s = lax.axis_index("s")
        wid = c * jnp.int32(NS) + s
        row0 = s * jnp.int32(RT)
        pltpu.sync_copy(zeros_hbm, acc.at[pl.ds(row0, RT)])
        plsc.subcore_barrier()

        for t in range(NSTAGE):
            c0 = jnp.int32(t * SCHUNK)
            pltpu.sync_copy(src_hbm.at[wid, pl.ds(c0, SCHUNK)], src_v)
            pltpu.sync_copy(dst_hbm.at[wid, pl.ds(c0, SCHUNK)], dst_v)
            for b in range(NBUF):
                pltpu.async_copy(
                    tab_hbm.at[src_v.at[jnp.int32(b)]], rows[b], sems[b])

            def body(i, carry):
                j0 = i * jnp.int32(NBUF)
                for b in range(NBUF):
                    j = j0 + jnp.int32(b)
                    pltpu.make_async_copy(
                        tab_hbm.at[src_v.at[j]], rows[b], sems[b]).wait()
                    pltpu.sync_copy(rows[b], acc.at[dst_v.at[j]], add=True)
                    pltpu.async_copy(
                        tab_hbm.at[src_v.at[j + jnp.int32(NBUF)]],
                        rows[b], sems[b])
                return carry

            lax.fori_loop(jnp.int32(0), jnp.int32(SCHUNK // NBUF - 1), body,
                          jnp.int32(0))
            jlast = jnp.int32(SCHUNK - NBUF)
            for b in range(NBUF):
                j = jlast + jnp.int32(b)
                pltpu.make_async_copy(
                    tab_hbm.at[src_v.at[j]], rows[b], sems[b]).wait()
                pltpu.sync_copy(rows[b], acc.at[dst_v.at[j]], add=True)

        plsc.subcore_barrier()
        pltpu.sync_copy(acc.at[pl.ds(row0, RT)], out_hbm.at[c, pl.ds(row0, RT)])

    return pl.kernel(agg, out_type=out_type, mesh=mesh, scratch_types=scratch)


@functools.lru_cache(maxsize=None)
def _make_deg():
    mesh = plsc.VectorSubcoreMesh(
        core_axis_name="c", subcore_axis_name="s", num_cores=NC, num_subcores=NS)

    out_type = [jax.ShapeDtypeStruct((NC, N_PAD, D), jnp.float32)]
    scratch = [
        pltpu.VMEM((NCHUNK, CH), jnp.int32),
        pltpu.VMEM((CH, D), jnp.float32),
        pltpu.VMEM_SHARED((N_PAD, D), jnp.float32),
    ]

    def deg(dst_hbm, zeros_hbm, ones_hbm, out_hbm, dst_v, ones_v, acc):
        c = lax.axis_index("c")
        s = lax.axis_index("s")
        wid = c * jnp.int32(NS) + s
        row0 = s * jnp.int32(RT)
        pltpu.sync_copy(zeros_hbm, acc.at[pl.ds(row0, RT)])
        pltpu.sync_copy(dst_hbm.at[wid], dst_v)
        pltpu.sync_copy(ones_hbm, ones_v)
        plsc.subcore_barrier()

        def body(j, carry):
            pltpu.sync_copy(ones_v, acc.at[dst_v.at[j]], add=True)
            return carry

        lax.fori_loop(jnp.int32(0), jnp.int32(NCHUNK), body, jnp.int32(0))
        plsc.subcore_barrier()
        pltpu.sync_copy(acc.at[pl.ds(row0, RT)],
                        out_hbm.at[c, pl.ds(row0, RT)])

    return pl.kernel(deg, out_type=out_type, mesh=mesh, scratch_types=scratch)


def _make_layer(relu, emit_inv):
    B = 1000
    grid = (N // B,)

    def body(*refs):
        if emit_inv:
            x_ref, acc_ref, d0_ref, d1_ref, ws_ref, wn_ref, b_ref, o_ref, inv_ref = refs
            deg = d0_ref[:, 0:1] + d1_ref[:, 0:1]
            inv = 1.0 / jnp.maximum(deg, 1.0)
            inv_ref[...] = jnp.broadcast_to(inv, (B, 8))
        else:
            x_ref, acc_ref, inv8_ref, ws_ref, wn_ref, b_ref, o_ref = refs
            inv = inv8_ref[:, 0:1]
        feats = acc_ref[0] + acc_ref[1]
        nb = feats * inv
        h = (jnp.dot(x_ref[...], ws_ref[...], preferred_element_type=jnp.float32)
             + jnp.dot(nb, wn_ref[...], preferred_element_type=jnp.float32)
             + b_ref[...])
        if relu:
            h = jnp.maximum(h, 0.0)
        o_ref[...] = h

    z = np.int32(0)
    x_spec = pl.BlockSpec((B, D), lambda i: (i, z))
    acc_spec = pl.BlockSpec((NC, B, D), lambda i: (z, i, z))
    w_spec = pl.BlockSpec((D, D), lambda i: (z, z))
    b_spec = pl.BlockSpec((1, D), lambda i: (z, z))
    inv_spec = pl.BlockSpec((B, 8), lambda i: (i, z))

    in_specs = ([x_spec, acc_spec]
                + ([inv_spec, inv_spec] if emit_inv else [inv_spec])
                + [w_spec, w_spec, b_spec])
    out_shape = [jax.ShapeDtypeStruct((N, D), jnp.float32)]
    out_specs = [x_spec]
    if emit_inv:
        out_shape.append(jax.ShapeDtypeStruct((N, 8), jnp.float32))
        out_specs.append(inv_spec)

    return pl.pallas_call(
        body, grid=grid, in_specs=in_specs,
        out_specs=out_specs if len(out_specs) > 1 else out_specs[0],
        out_shape=out_shape if len(out_shape) > 1 else out_shape[0])


_layer0 = _make_layer(relu=True, emit_inv=True)
_layer1 = _make_layer(relu=True, emit_inv=False)
_layer2 = _make_layer(relu=False, emit_inv=False)


def _fold_bn(Ws, Wn, b, gamma, beta, eps=1e-5):
    s = gamma * np.float32(1.0 / np.sqrt(1.0 + eps))
    return Ws * s[None, :], Wn * s[None, :], (b * s + beta).reshape(1, D)


def kernel(x, edge_index, W_self0, W_neigh0, b0, gamma0, beta0,
           W_self1, W_neigh1, b1, gamma1, beta1, W_self2, W_neigh2, b2):
    x = x.astype(jnp.float32)
    src = edge_index[0].astype(jnp.int32)
    dst = edge_index[1].astype(jnp.int32)
    pad = (np.arange(E_PAD - E, dtype=np.int32) % np.int32(NDUMMY)) + np.int32(N)
    src_p = jnp.concatenate([src, pad]).reshape(NW, NCHUNK, CH)
    dst_p = jnp.concatenate([dst, pad]).reshape(NW, NCHUNK, CH)

    row_pad = np.zeros((N_PAD - N, D), np.float32)
    z = np.zeros((RT, D), np.float32)

    Ws0, Wn0, B0 = _fold_bn(W_self0, W_neigh0, b0, gamma0, beta0)
    Ws1, Wn1, B1 = _fold_bn(W_self1, W_neigh1, b1, gamma1, beta1)
    B2 = b2.reshape(1, D)

    x_pad = jnp.concatenate([x, row_pad], axis=0)
    (acc0,) = _make_agg()(x_pad, src_p, dst_p, z)
    ones = np.ones((CH, D), np.float32)
    (degp,) = _make_deg()(dst_p, z, ones)
    d0 = degp[0, :N, 0:8]
    d1 = degp[1, :N, 0:8]
    h1, inv8 = _layer0(x, acc0, d0, d1, Ws0, Wn0, B0)

    (acc1,) = _make_agg()(jnp.concatenate([h1, row_pad], axis=0), src_p, dst_p, z)
    h2 = _layer1(h1, acc1, inv8, Ws1, Wn1, B1)

    (acc2,) = _make_agg()(jnp.concatenate([h2, row_pad], axis=0), src_p, dst_p, z)
    h3 = _layer2(h2, acc2, inv8, W_self2, W_neigh2, B2)
    return h3
